# baseline (speedup 1.0000x reference)
import functools

import jax
import jax.numpy as jnp
from jax import lax
from jax.experimental import pallas as pl
from jax.experimental.pallas import tpu as pltpu
from jax.experimental.pallas import tpu_sc as plsc

_NC = 2
_NS = 16
_NW = _NC * _NS

_CHUNK = 1600
_NBUF = 2


def _transpose_pack(table_t, aux_pk):
  emb_dim, vocab = table_t.shape
  n_pk = vocab * emb_dim // 128
  n_full = vocab // 128
  n_aux = aux_pk.shape[0]
  per_w = (n_full + _NW - 1) // _NW
  mesh = plsc.VectorSubcoreMesh(core_axis_name="c", subcore_axis_name="s")

  @functools.partial(
      pl.kernel,
      out_type=jax.ShapeDtypeStruct((n_pk, 128), jnp.float32),
      mesh=mesh,
      scratch_types=(
          [pltpu.VMEM((emb_dim, 129), jnp.float32) for _ in range(2)]
          + [pltpu.VMEM((32, 128), jnp.float32) for _ in range(2)]
          + [pltpu.SemaphoreType.DMA for _ in range(4)]
      ),
      compiler_params=pltpu.CompilerParams(needs_layout_passes=False),
  )
  def k(tt_hbm, aux_hbm, pk_hbm, *refs):
    b_v = refs[0:2]
    p_v = refs[2:4]
    ssem = refs[4:6]
    wsem = refs[6:8]
    wid = lax.axis_index("s") * _NC + lax.axis_index("c")

    def vt_of(i):
      return i * _NW + wid

    def fire_stage(i, s):
      pltpu.async_copy(
          tt_hbm.at[:, pl.ds(vt_of(i) * 128, 128)],
          b_v[s].at[:, pl.ds(0, 128)], ssem[s])

    def wait_stage(s):
      pltpu.make_async_copy(
          tt_hbm.at[:, pl.ds(0, 128)],
          b_v[s].at[:, pl.ds(0, 128)], ssem[s]).wait()

    def wait_write(s):
      pltpu.make_async_copy(
          p_v[s], pk_hbm.at[pl.ds(0, 32), :], wsem[s]).wait()

    def compact_write(i, s):
      for r in range(32):
        for l16 in range(8):
          row_ids = lax.iota(jnp.int32, 16) + 16 * (l16 % 2)
          col_ids = jnp.full((16,), 4 * r + l16 // 2, jnp.int32)
          vals = plsc.load_gather(b_v[s], [row_ids, col_ids])
          p_v[s][r, pl.ds(16 * l16, 16)] = vals
      pltpu.async_copy(
          p_v[s], pk_hbm.at[pl.ds(vt_of(i) * 32, 32), :], wsem[s])

    @pl.when(vt_of(0) < n_full)
    def _():
      fire_stage(0, 0)

    def body(i, carry):
      for s in (0, 1):
        j = 2 * i + s

        @pl.when(vt_of(j + 1) < n_full)
        def _():
          fire_stage(j + 1, 1 - s)

        @pl.when((j >= 2) & (vt_of(j) < n_full + 2 * _NW))
        def _():
          wait_write(s)

        @pl.when(vt_of(j) < n_full)
        def _():
          wait_stage(s)
          compact_write(j, s)
      return carry

    nsteps = (per_w + 1) // 2
    lax.fori_loop(0, nsteps, body, 0)
    for j in (2 * nsteps - 2, 2 * nsteps - 1):
      @pl.when(vt_of(j) < n_full)
      def _():
        wait_write(j % 2)

    @pl.when(wid == 0)
    def _():
      pltpu.sync_copy(aux_hbm, p_v[0].at[pl.ds(0, n_aux)])
      pltpu.sync_copy(p_v[0].at[pl.ds(0, n_aux)],
                      pk_hbm.at[pl.ds(n_pk - n_aux, n_aux), :])

  return k(table_t, aux_pk)


def _emb_gather(table, idx):
  total = idx.shape[0]
  b_per_w = total // _NW
  nchunk = b_per_w // _CHUNK
  emb_dim = table.shape[1]
  mesh = plsc.VectorSubcoreMesh(core_axis_name="c", subcore_axis_name="s")

  scratch = (
      [pltpu.VMEM((_CHUNK,), jnp.int32) for _ in range(_NBUF)]
      + [pltpu.VMEM((_CHUNK, emb_dim), jnp.float32) for _ in range(_NBUF)]
      + [pltpu.SemaphoreType.DMA for _ in range(3 * _NBUF)]
  )

  @functools.partial(
      pl.kernel,
      out_type=jax.ShapeDtypeStruct((total // 50, 50, emb_dim), jnp.float32),
      mesh=mesh,
      scratch_types=scratch,
      compiler_params=pltpu.CompilerParams(use_tc_tiling_on_sc=False),
  )
  def k(table_hbm, idx_hbm, out_3d, *refs):
    idx_bufs = refs[0:_NBUF]
    row_bufs = refs[_NBUF:2 * _NBUF]
    sem_i = refs[2 * _NBUF:2 * _NBUF + _NBUF]
    sem_g = refs[3 * _NBUF:3 * _NBUF + _NBUF]
    sem_o = refs[4 * _NBUF:4 * _NBUF + _NBUF]

    wid = lax.axis_index("s") * _NC + lax.axis_index("c")
    base = wid * b_per_w

    def idx_copy(c):
      b = c % _NBUF
      return pltpu.async_copy(
          idx_hbm.at[pl.ds(base + c * _CHUNK, _CHUNK)], idx_bufs[b], sem_i[b])

    def gather(c):
      b = c % _NBUF
      return pltpu.async_copy(table_hbm.at[idx_bufs[b]], row_bufs[b], sem_g[b])

    batches_per_chunk = _CHUNK // 50

    def out_copy(c):
      b = c % _NBUF
      b0 = (base + c * _CHUNK) // 50
      return [
          pltpu.async_copy(
              row_bufs[b].at[pl.ds(j * 50, 50)], out_3d.at[b0 + j], sem_o[b])
          for j in range(batches_per_chunk)
      ]

    cp_i, cp_g, cp_o = {}, {}, {}
    for t in range(nchunk + 2):
      c = t - 2
      if 0 <= c < nchunk:
        cp_g[c].wait()
        cp_o[c] = out_copy(c)
      c = t - 1
      if 0 <= c < nchunk:
        cp_i[c].wait()
        if c - _NBUF >= 0:
          for d in cp_o.pop(c - _NBUF):
            d.wait()
        cp_g[c] = gather(c)
      if t < nchunk:
        cp_i[t] = idx_copy(t)
    for c in sorted(cp_o):
      for d in cp_o[c]:
        d.wait()

  return k(table, idx)


def kernel(x, table):
  nrow, dim = table.shape
  idx = x.reshape(-1).astype(jnp.int32)
  tail = (nrow // 128) * 128
  aux_pk = table[tail:].reshape((nrow - tail) * dim // 128, 128)
  table_pk = _transpose_pack(jnp.transpose(table), aux_pk)
  return _emb_gather(table_pk.reshape(nrow, dim), idx)


# speedup vs baseline: 1.4543x; 1.4543x over previous
import functools

import jax
import jax.numpy as jnp
from jax import lax
from jax.experimental import pallas as pl
from jax.experimental.pallas import tpu as pltpu
from jax.experimental.pallas import tpu_sc as plsc

_NC = 2
_NS = 16
_NW = _NC * _NS

_CHUNK = 1600
_NBUF = 2


def _emb_gather(table, idx):
  total = idx.shape[0]
  b_per_w = total // _NW
  nchunk = b_per_w // _CHUNK
  emb_dim = table.shape[1]
  mesh = plsc.VectorSubcoreMesh(core_axis_name="c", subcore_axis_name="s")

  scratch = (
      [pltpu.VMEM((_CHUNK,), jnp.int32) for _ in range(_NBUF)]
      + [pltpu.VMEM((_CHUNK, emb_dim), jnp.float32) for _ in range(_NBUF)]
      + [pltpu.SemaphoreType.DMA for _ in range(3 * _NBUF)]
  )

  @functools.partial(
      pl.kernel,
      out_type=jax.ShapeDtypeStruct((total // 50, 50, emb_dim), jnp.float32),
      mesh=mesh,
      scratch_types=scratch,
      compiler_params=pltpu.CompilerParams(use_tc_tiling_on_sc=False),
  )
  def k(table_hbm, idx_hbm, out_3d, *refs):
    idx_bufs = refs[0:_NBUF]
    row_bufs = refs[_NBUF:2 * _NBUF]
    sem_i = refs[2 * _NBUF:2 * _NBUF + _NBUF]
    sem_g = refs[3 * _NBUF:3 * _NBUF + _NBUF]
    sem_o = refs[4 * _NBUF:4 * _NBUF + _NBUF]

    wid = lax.axis_index("s") * _NC + lax.axis_index("c")
    base = wid * b_per_w

    def idx_copy(c):
      b = c % _NBUF
      return pltpu.async_copy(
          idx_hbm.at[pl.ds(base + c * _CHUNK, _CHUNK)], idx_bufs[b], sem_i[b])

    def gather(c):
      b = c % _NBUF
      return pltpu.async_copy(table_hbm.at[idx_bufs[b]], row_bufs[b], sem_g[b])

    batches_per_chunk = _CHUNK // 50

    def out_copy(c):
      b = c % _NBUF
      b0 = (base + c * _CHUNK) // 50
      return [
          pltpu.async_copy(
              row_bufs[b].at[pl.ds(j * 50, 50)], out_3d.at[b0 + j], sem_o[b])
          for j in range(batches_per_chunk)
      ]

    cp_i, cp_g, cp_o = {}, {}, {}
    for t in range(nchunk + 2):
      c = t - 2
      if 0 <= c < nchunk:
        cp_g[c].wait()
        cp_o[c] = out_copy(c)
      c = t - 1
      if 0 <= c < nchunk:
        cp_i[c].wait()
        if c - _NBUF >= 0:
          for d in cp_o.pop(c - _NBUF):
            d.wait()
        cp_g[c] = gather(c)
      if t < nchunk:
        cp_i[t] = idx_copy(t)
    for c in sorted(cp_o):
      for d in cp_o[c]:
        d.wait()

  return k(table, idx)


def kernel(x, table):
  idx = x.reshape(-1).astype(jnp.int32)
  return _emb_gather(table, idx)
